# baseline (speedup 1.0000x reference)
import jax
import jax.numpy as jnp
from jax import lax
from jax.experimental import pallas as pl
from jax.experimental.pallas import tpu as pltpu

D_MODEL = 1024
NUM_TILES = 64
TILES_PER_CLUSTER = 8
GRID_SIZE = 64
N_CLUSTERS = NUM_TILES // TILES_PER_CLUSTER
TG = NUM_TILES * GRID_SIZE
CHUNK = TG // N_CLUSTERS
N_TOK = 2048
BLK = 256
NBLK = N_TOK // BLK

_NEG = -1e30
_F32 = jnp.float32


def _first_argmax(vals, maxv, width):
    col = lax.broadcasted_iota(jnp.int32, vals.shape, 1)
    cand = jnp.where(vals >= maxv, col, jnp.int32(width))
    return jnp.min(cand, axis=1, keepdims=True)


def _dispatch_body(x_ref, wc_ref, p_ref,
                   xs_ref, tidx_ref, gate_ref, dst_ref, cid_ref, act_ref):
    xb = x_ref[...]

    cl = lax.dot_general(xb, wc_ref[...], (((1,), (1,)), ((), ())),
                         preferred_element_type=_F32)
    cmax = jnp.max(cl, axis=1, keepdims=True)
    csum = jnp.sum(jnp.exp(cl - cmax), axis=1, keepdims=True)
    c_idx = _first_argmax(cl, cmax, N_CLUSTERS)
    tl = lax.dot_general(xb, p_ref[...], (((1,), (1,)), ((), ())),
                         preferred_element_type=_F32)
    tcol = lax.broadcasted_iota(jnp.int32, tl.shape, 1) // TILES_PER_CLUSTER
    tlm = jnp.where(tcol == c_idx, tl, _NEG)
    tmax = jnp.max(tlm, axis=1, keepdims=True)
    tsum = jnp.sum(jnp.exp(tlm - tmax), axis=1, keepdims=True)
    t_idx = _first_argmax(tlm, tmax, NUM_TILES)
    gate = 1.0 / (csum * tsum)

    ccol = lax.broadcasted_iota(jnp.int32, (N_TOK, N_CLUSTERS), 1)
    onehot = (ccol == c_idx).astype(jnp.bfloat16)
    counts = jnp.sum(onehot.astype(_F32), axis=0, keepdims=True)
    tri_r = lax.broadcasted_iota(jnp.int32, (N_CLUSTERS, N_CLUSTERS), 0)
    tri_c = lax.broadcasted_iota(jnp.int32, (N_CLUSTERS, N_CLUSTERS), 1)
    tri = (tri_r < tri_c).astype(_F32)
    offs = lax.dot_general(counts, tri, (((1,), (0,)), ((), ())),
                           preferred_element_type=_F32)
    row_i = lax.broadcasted_iota(jnp.int32, (N_TOK, N_TOK), 0)
    col_i = lax.broadcasted_iota(jnp.int32, (N_TOK, N_TOK), 1)
    ltri = (col_i < row_i).astype(jnp.bfloat16)
    prefix = lax.dot_general(ltri, onehot, (((1,), (0,)), ((), ())),
                             preferred_element_type=_F32)
    onef = onehot.astype(_F32)
    rank = jnp.sum(prefix * onef, axis=1, keepdims=True)
    base = jnp.sum(offs * onef, axis=1, keepdims=True)
    dst = base + rank
    dst_ref[...] = dst

    dsti = dst.astype(jnp.int32)
    perm = (col_i == dsti).astype(jnp.bfloat16)
    xh = (xb * (1.0 / (D_MODEL ** 0.5))).astype(jnp.bfloat16)
    xs_ref[...] = lax.dot_general(perm, xh, (((0,), (0,)), ((), ())),
                                  preferred_element_type=_F32
                                  ).astype(jnp.bfloat16)
    tidx_ref[...] = lax.dot_general(perm, t_idx.astype(jnp.bfloat16),
                                    (((0,), (0,)), ((), ())),
                                    preferred_element_type=_F32)
    gate_ref[...] = lax.dot_general(perm, gate.astype(jnp.bfloat16),
                                    (((0,), (0,)), ((), ())),
                                    preferred_element_type=_F32)

    cum_next = offs + counts
    brow = lax.broadcasted_iota(jnp.int32, (NBLK, N_CLUSTERS), 0)
    lo = (brow * BLK).astype(_F32)
    hi = lo + float(BLK)
    c0 = jnp.sum((cum_next <= lo).astype(jnp.int32), axis=1, keepdims=True)
    c0n = jnp.sum((cum_next <= hi).astype(jnp.int32), axis=1, keepdims=True)
    overlap = jnp.logical_and(offs < hi, cum_next > lo)
    nb = jnp.sum(overlap.astype(jnp.int32), axis=1, keepdims=True)
    jcol = lax.broadcasted_iota(jnp.int32, (NBLK, N_CLUSTERS), 1)
    act = jcol < nb
    cid = jnp.where(act, c0 + jcol, jnp.minimum(c0n, N_CLUSTERS - 1))
    cid_ref[...] = jnp.minimum(cid, N_CLUSTERS - 1)
    act_ref[...] = act.astype(jnp.int32)


def _lookup_body(cid_ref, act_ref, xs_ref, tidx_ref, gate_ref,
                 k_ref, v_ref, dout_ref, dacc_ref):
    g = pl.program_id(0)
    b = g // N_CLUSTERS
    j = g - b * N_CLUSTERS

    @pl.when(j == 0)
    def _init():
        dacc_ref[...] = jnp.zeros_like(dacc_ref)

    @pl.when(act_ref[b, j] == 1)
    def _compute():
        c = cid_ref[b, j]
        xh = xs_ref[...]
        kc = k_ref[...].astype(jnp.bfloat16)
        gl = lax.dot_general(xh, kc, (((1,), (1,)), ((), ())),
                             preferred_element_type=_F32)
        tcol = (lax.broadcasted_iota(jnp.int32, gl.shape, 1) // GRID_SIZE
                + c * TILES_PER_CLUSTER)
        tid = tidx_ref[...].astype(jnp.int32)
        pr = jnp.where(tcol == tid, jnp.exp(jnp.minimum(gl, 60.0)), 0.0)
        s = jnp.sum(pr, axis=1, keepdims=True)
        y = lax.dot_general(pr.astype(jnp.bfloat16),
                            v_ref[...].astype(jnp.bfloat16),
                            (((1,), (0,)), ((), ())),
                            preferred_element_type=_F32)
        w = gate_ref[...] / jnp.maximum(s, 1e-30)
        dacc_ref[...] += w * y

    @pl.when(j == N_CLUSTERS - 1)
    def _emit():
        dout_ref[...] = dacc_ref[...].astype(jnp.bfloat16)


def _undispatch_body(x_ref, dst_ref, ds_ref, o_ref):
    col_i = lax.broadcasted_iota(jnp.int32, (N_TOK, N_TOK), 1)
    perm = (col_i == dst_ref[...].astype(jnp.int32)).astype(jnp.bfloat16)
    delta = lax.dot_general(perm, ds_ref[...], (((1,), (0,)), ((), ())),
                            preferred_element_type=_F32)
    o_ref[...] = x_ref[...] + delta


@jax.jit
def kernel(x, Wc, P, Kt, Vt):
    n, d = x.shape
    k2 = Kt.reshape(TG, d)
    v2 = Vt.reshape(TG, d)

    xs, tidx_s, gate_s, dst, cid, act = pl.pallas_call(
        _dispatch_body,
        grid=(1,),
        in_specs=[
            pl.BlockSpec((n, d), lambda i: (0, 0)),
            pl.BlockSpec((N_CLUSTERS, d), lambda i: (0, 0)),
            pl.BlockSpec((NUM_TILES, d), lambda i: (0, 0)),
        ],
        out_specs=[
            pl.BlockSpec((n, d), lambda i: (0, 0)),
            pl.BlockSpec((n, 1), lambda i: (0, 0)),
            pl.BlockSpec((n, 1), lambda i: (0, 0)),
            pl.BlockSpec((n, 1), lambda i: (0, 0)),
            pl.BlockSpec((NBLK, N_CLUSTERS), lambda i: (0, 0)),
            pl.BlockSpec((NBLK, N_CLUSTERS), lambda i: (0, 0)),
        ],
        out_shape=[
            jax.ShapeDtypeStruct((n, d), jnp.bfloat16),
            jax.ShapeDtypeStruct((n, 1), jnp.float32),
            jax.ShapeDtypeStruct((n, 1), jnp.float32),
            jax.ShapeDtypeStruct((n, 1), jnp.float32),
            jax.ShapeDtypeStruct((NBLK, N_CLUSTERS), jnp.int32),
            jax.ShapeDtypeStruct((NBLK, N_CLUSTERS), jnp.int32),
        ],
        compiler_params=pltpu.CompilerParams(
            dimension_semantics=("arbitrary",),
        ),
    )(x, Wc, P)

    delta_s = pl.pallas_call(
        _lookup_body,
        grid_spec=pltpu.PrefetchScalarGridSpec(
            num_scalar_prefetch=2,
            grid=(NBLK * N_CLUSTERS,),
            in_specs=[
                pl.BlockSpec((BLK, d), lambda g, cid, act: (g // N_CLUSTERS, 0)),
                pl.BlockSpec((BLK, 1), lambda g, cid, act: (g // N_CLUSTERS, 0)),
                pl.BlockSpec((BLK, 1), lambda g, cid, act: (g // N_CLUSTERS, 0)),
                pl.BlockSpec(
                    (CHUNK, d),
                    lambda g, cid, act: (cid[g // N_CLUSTERS,
                                             g % N_CLUSTERS], 0)),
                pl.BlockSpec(
                    (CHUNK, d),
                    lambda g, cid, act: (cid[g // N_CLUSTERS,
                                             g % N_CLUSTERS], 0)),
            ],
            out_specs=pl.BlockSpec((BLK, d),
                                   lambda g, cid, act: (g // N_CLUSTERS, 0)),
            scratch_shapes=[pltpu.VMEM((BLK, d), jnp.float32)],
        ),
        out_shape=jax.ShapeDtypeStruct((n, d), jnp.bfloat16),
        compiler_params=pltpu.CompilerParams(
            dimension_semantics=("arbitrary",),
        ),
    )(cid, act, xs, tidx_s, gate_s, k2, v2)

    return pl.pallas_call(
        _undispatch_body,
        grid=(1,),
        in_specs=[
            pl.BlockSpec((n, d), lambda i: (0, 0)),
            pl.BlockSpec((n, 1), lambda i: (0, 0)),
            pl.BlockSpec((n, d), lambda i: (0, 0)),
        ],
        out_specs=pl.BlockSpec((n, d), lambda i: (0, 0)),
        out_shape=jax.ShapeDtypeStruct((n, d), jnp.float32),
        compiler_params=pltpu.CompilerParams(
            dimension_semantics=("arbitrary",),
        ),
    )(x, dst, delta_s)


# speedup vs baseline: 1.1817x; 1.1817x over previous
import jax
import jax.numpy as jnp
from jax import lax
from jax.experimental import pallas as pl
from jax.experimental.pallas import tpu as pltpu

D_MODEL = 1024
NUM_TILES = 64
TILES_PER_CLUSTER = 8
GRID_SIZE = 64
N_CLUSTERS = NUM_TILES // TILES_PER_CLUSTER
TG = NUM_TILES * GRID_SIZE
CHUNK = TG // N_CLUSTERS
N_TOK = 2048
BLK = 256
NBLK = N_TOK // BLK
NSEG = 16
GRP = 256
NGRP = N_TOK // GRP

_NEG = -1e30
_F32 = jnp.float32


def _first_argmax(vals, maxv, width):
    col = lax.broadcasted_iota(jnp.int32, vals.shape, 1)
    cand = jnp.where(vals >= maxv, col, jnp.int32(width))
    return jnp.min(cand, axis=1, keepdims=True)


def _body(x_ref, wc_ref, p_ref, k_hbm, v_hbm, o_ref,
          perm_ref, xs_ref, tidx_ref, gate_ref, delta_ref,
          kbuf, vbuf, sched_v, sched_s, ksem, vsem, ssem):
    g = pl.program_id(0)

    def _start_chunk(c, p):
        kcp = pltpu.make_async_copy(
            k_hbm.at[pl.ds(c * CHUNK, CHUNK), :],
            kbuf.at[pl.ds(p * CHUNK, CHUNK), :], ksem)
        vcp = pltpu.make_async_copy(
            v_hbm.at[pl.ds(c * CHUNK, CHUNK), :],
            vbuf.at[pl.ds(p * CHUNK, CHUNK), :], vsem)
        kcp.start()
        vcp.start()

    def _wait_chunk(c, p):
        pltpu.make_async_copy(
            k_hbm.at[pl.ds(c * CHUNK, CHUNK), :],
            kbuf.at[pl.ds(p * CHUNK, CHUNK), :], ksem).wait()
        pltpu.make_async_copy(
            v_hbm.at[pl.ds(c * CHUNK, CHUNK), :],
            vbuf.at[pl.ds(p * CHUNK, CHUNK), :], vsem).wait()

    @pl.when(g == 0)
    def _dispatch():
        xb = x_ref[...]

        cl = lax.dot_general(xb, wc_ref[...], (((1,), (1,)), ((), ())),
                             preferred_element_type=_F32)
        cmax = jnp.max(cl, axis=1, keepdims=True)
        csum = jnp.sum(jnp.exp(cl - cmax), axis=1, keepdims=True)
        c_idx = _first_argmax(cl, cmax, N_CLUSTERS)
        tl = lax.dot_general(xb, p_ref[...], (((1,), (1,)), ((), ())),
                             preferred_element_type=_F32)
        tcol = (lax.broadcasted_iota(jnp.int32, tl.shape, 1)
                // TILES_PER_CLUSTER)
        tlm = jnp.where(tcol == c_idx, tl, _NEG)
        tmax = jnp.max(tlm, axis=1, keepdims=True)
        tsum = jnp.sum(jnp.exp(tlm - tmax), axis=1, keepdims=True)
        t_idx = _first_argmax(tlm, tmax, NUM_TILES)
        gate = 1.0 / (csum * tsum)

        ccol = lax.broadcasted_iota(jnp.int32, (N_TOK, N_CLUSTERS), 1)
        onehot = (ccol == c_idx).astype(jnp.bfloat16)
        onef = onehot.astype(_F32)
        r256 = lax.broadcasted_iota(jnp.int32, (GRP, GRP), 0)
        c256 = lax.broadcasted_iota(jnp.int32, (GRP, GRP), 1)
        ltri = (c256 < r256).astype(jnp.bfloat16)
        chunks = []
        gcs = []
        for q in range(NGRP):
            oh_q = onehot[q * GRP:(q + 1) * GRP, :]
            chunks.append(lax.dot_general(
                ltri, oh_q, (((1,), (0,)), ((), ())),
                preferred_element_type=_F32))
            gcs.append(jnp.sum(oh_q.astype(_F32), axis=0, keepdims=True))
        prefix = jnp.concatenate(chunks, axis=0)
        acc = jnp.zeros((1, N_CLUSTERS), _F32)
        grows = []
        for q in range(NGRP):
            grows.append(acc)
            acc = acc + gcs[q]
        counts = acc
        base_g = jnp.concatenate(
            [jnp.broadcast_to(grows[q], (GRP, N_CLUSTERS))
             for q in range(NGRP)], axis=0)
        offs_cols = [jnp.zeros((1, 1), _F32)]
        for c in range(1, N_CLUSTERS):
            offs_cols.append(offs_cols[-1] + counts[:, c - 1:c])
        offs = jnp.concatenate(offs_cols, axis=1)
        dst = jnp.sum(onef * (offs + base_g + prefix), axis=1,
                      keepdims=True)
        dsti = dst.astype(jnp.int32)

        for st in range(N_TOK // 256):
            colt = (lax.broadcasted_iota(jnp.int32, (N_TOK, 256), 1)
                    + st * 256)
            perm_ref[:, st * 256:(st + 1) * 256] = (
                (colt == dsti).astype(jnp.bfloat16))
        permm = perm_ref[...]
        xh = (xb * (1.0 / (D_MODEL ** 0.5))).astype(jnp.bfloat16)
        xs_ref[...] = lax.dot_general(
            permm, xh, (((0,), (0,)), ((), ())),
            preferred_element_type=_F32).astype(jnp.bfloat16)
        tidx_ref[...] = lax.dot_general(
            permm, t_idx.astype(jnp.bfloat16), (((0,), (0,)), ((), ())),
            preferred_element_type=_F32).astype(jnp.int32)
        gate_ref[...] = lax.dot_general(
            permm, gate.astype(jnp.bfloat16), (((0,), (0,)), ((), ())),
            preferred_element_type=_F32)
        delta_ref[...] = jnp.zeros_like(delta_ref)

        cum_next = offs + counts
        brow = lax.broadcasted_iota(jnp.int32, (NBLK, N_CLUSTERS), 0)
        lo = (brow * BLK).astype(_F32)
        hi = lo + float(BLK)
        c0 = jnp.sum((cum_next <= lo).astype(_F32), axis=1, keepdims=True)
        overlap = jnp.logical_and(offs < hi, cum_next > lo)
        nb = jnp.sum(overlap.astype(_F32), axis=1, keepdims=True)
        tri_r = lax.broadcasted_iota(jnp.int32, (NBLK, NBLK), 0)
        tri_c = lax.broadcasted_iota(jnp.int32, (NBLK, NBLK), 1)
        ltri_b = (tri_r > tri_c).astype(_F32)
        itri_b = (tri_r >= tri_c).astype(_F32)
        cum_ex = lax.dot_general(ltri_b, nb, (((1,), (0,)), ((), ())),
                                 preferred_element_type=_F32)
        cum_in = lax.dot_general(itri_b, nb, (((1,), (0,)), ((), ())),
                                 preferred_element_type=_F32)
        ones_s = jnp.ones((NSEG, 1), _F32)
        cumin_r = lax.dot_general(ones_s, cum_in, (((1,), (1,)), ((), ())),
                                  preferred_element_type=_F32)
        cumex_r = lax.dot_general(ones_s, cum_ex, (((1,), (1,)), ((), ())),
                                  preferred_element_type=_F32)
        c0_r = lax.dot_general(ones_s, c0, (((1,), (1,)), ((), ())),
                               preferred_element_type=_F32)
        s_col = lax.broadcasted_iota(jnp.int32, (NSEG, 1), 0).astype(_F32)
        b_of_s = jnp.sum((cumin_r <= s_col).astype(_F32), axis=1,
                         keepdims=True)
        bcol = lax.broadcasted_iota(jnp.int32, (NSEG, NBLK), 1).astype(_F32)
        oh_b = (bcol == b_of_s).astype(_F32)
        cumex_s = jnp.sum(oh_b * cumex_r, axis=1, keepdims=True)
        c0_s = jnp.sum(oh_b * c0_r, axis=1, keepdims=True)
        j_s = s_col - cumex_s
        total = jnp.sum(nb)
        s_act = s_col < total
        cid_f = jnp.clip(c0_s + j_s, 0.0, float(N_CLUSTERS - 1))
        cid_f = jnp.where(s_act, cid_f, float(N_CLUSTERS - 1))
        sblk = jnp.clip(b_of_s, 0.0, float(NBLK - 1))
        prev_cid = jnp.concatenate(
            [jnp.full((1, 1), -1.0, _F32), cid_f[:NSEG - 1, :]], axis=0)
        snew = jnp.logical_and(s_act, cid_f != prev_cid)
        sr = lax.broadcasted_iota(jnp.int32, (NSEG, NSEG), 0)
        sc = lax.broadcasted_iota(jnp.int32, (NSEG, NSEG), 1)
        itri_s = (sr >= sc).astype(jnp.bfloat16)
        nfetch = lax.dot_general(itri_s, snew.astype(jnp.bfloat16),
                                 (((1,), (0,)), ((), ())),
                                 preferred_element_type=_F32)
        sbuf = (nfetch.astype(jnp.int32) + 1) % 2
        sched = jnp.concatenate([
            sblk.astype(jnp.int32), cid_f.astype(jnp.int32),
            s_act.astype(jnp.int32), snew.astype(jnp.int32), sbuf,
            jnp.zeros((NSEG, 3), jnp.int32)], axis=1)
        sched_v[...] = sched
        cp = pltpu.make_async_copy(sched_v, sched_s, ssem)
        cp.start()
        cp.wait()
        _start_chunk(sched_s[0, 1], sched_s[0, 4])

    @pl.when(jnp.logical_and(g >= 1, g <= NSEG))
    def _lookup():
        seg = g - 1
        b = sched_s[seg, 0]
        c = sched_s[seg, 1]
        act = sched_s[seg, 2]
        new = sched_s[seg, 3]
        p = sched_s[seg, 4]

        @pl.when(new == 1)
        def _w():
            _wait_chunk(c, p)

        @pl.when(act == 1)
        def _compute():
            xh = xs_ref[pl.ds(b * BLK, BLK), :]
            kc = kbuf[pl.ds(p * CHUNK, CHUNK), :].astype(jnp.bfloat16)
            gl = lax.dot_general(xh, kc, (((1,), (1,)), ((), ())),
                                 preferred_element_type=_F32)
            tcol = (lax.broadcasted_iota(jnp.int32, gl.shape, 1) // GRID_SIZE
                    + c * TILES_PER_CLUSTER)
            tid = tidx_ref[pl.ds(b * BLK, BLK), :]
            pr = jnp.where(tcol == tid, jnp.exp(jnp.minimum(gl, 60.0)), 0.0)
            s = jnp.sum(pr, axis=1, keepdims=True)
            vc = vbuf[pl.ds(p * CHUNK, CHUNK), :].astype(jnp.bfloat16)
            y = lax.dot_general(pr.astype(jnp.bfloat16), vc,
                                (((1,), (0,)), ((), ())),
                                preferred_element_type=_F32)
            w = gate_ref[pl.ds(b * BLK, BLK), :] / jnp.maximum(s, 1e-30)
            delta_ref[pl.ds(b * BLK, BLK), :] += (w * y).astype(jnp.bfloat16)

        nxt = jnp.minimum(g, NSEG - 1)
        @pl.when(jnp.logical_and(g < NSEG, sched_s[nxt, 3] == 1))
        def _pf():
            _start_chunk(sched_s[nxt, 1], sched_s[nxt, 4])

    @pl.when(g == NSEG + 1)
    def _undispatch():
        back = lax.dot_general(perm_ref[...], delta_ref[...],
                               (((1,), (0,)), ((), ())),
                               preferred_element_type=_F32)
        o_ref[...] = x_ref[...] + back


@jax.jit
def kernel(x, Wc, P, Kt, Vt):
    n, d = x.shape
    k2 = Kt.reshape(TG, d)
    v2 = Vt.reshape(TG, d)
    return pl.pallas_call(
        _body,
        grid=(NSEG + 2,),
        in_specs=[
            pl.BlockSpec((n, d), lambda g: (0, 0)),
            pl.BlockSpec((N_CLUSTERS, d), lambda g: (0, 0)),
            pl.BlockSpec((NUM_TILES, d), lambda g: (0, 0)),
            pl.BlockSpec(memory_space=pltpu.MemorySpace.HBM),
            pl.BlockSpec(memory_space=pltpu.MemorySpace.HBM),
        ],
        out_specs=pl.BlockSpec((n, d), lambda g: (0, 0)),
        out_shape=jax.ShapeDtypeStruct((n, d), jnp.float32),
        scratch_shapes=[
            pltpu.VMEM((N_TOK, N_TOK), jnp.bfloat16),
            pltpu.VMEM((N_TOK, D_MODEL), jnp.bfloat16),
            pltpu.VMEM((N_TOK, 1), jnp.int32),
            pltpu.VMEM((N_TOK, 1), jnp.float32),
            pltpu.VMEM((N_TOK, D_MODEL), jnp.bfloat16),
            pltpu.VMEM((2 * CHUNK, D_MODEL), jnp.float32),
            pltpu.VMEM((2 * CHUNK, D_MODEL), jnp.float32),
            pltpu.VMEM((NSEG, 8), jnp.int32),
            pltpu.SMEM((NSEG, 8), jnp.int32),
            pltpu.SemaphoreType.DMA,
            pltpu.SemaphoreType.DMA,
            pltpu.SemaphoreType.DMA,
        ],
        compiler_params=pltpu.CompilerParams(
            dimension_semantics=("arbitrary",),
            vmem_limit_bytes=100 * 1024 * 1024,
        ),
    )(x, Wc, P, k2, v2)


# speedup vs baseline: 1.2358x; 1.0458x over previous
import jax
import jax.numpy as jnp
from jax import lax
from jax.experimental import pallas as pl
from jax.experimental.pallas import tpu as pltpu

D_MODEL = 1024
NUM_TILES = 64
TILES_PER_CLUSTER = 8
GRID_SIZE = 64
N_CLUSTERS = NUM_TILES // TILES_PER_CLUSTER
TG = NUM_TILES * GRID_SIZE
CHUNK = TG // N_CLUSTERS

_NEG = -1e30
_F32 = jnp.float32


def _first_argmax(vals, maxv, width):
    col = lax.broadcasted_iota(jnp.int32, vals.shape, 1)
    cand = jnp.where(vals >= maxv, col, jnp.int32(width))
    return jnp.min(cand, axis=1, keepdims=True)


def _body(x_ref, wc_ref, p_ref, k_ref, v_ref, o_ref,
          xh_ref, tidx_ref, gate_ref, yacc_ref, sacc_ref):
    c = pl.program_id(0)

    @pl.when(c == 0)
    def _routing():
        xb = x_ref[...]
        cl = lax.dot_general(xb, wc_ref[...], (((1,), (1,)), ((), ())),
                             preferred_element_type=_F32)
        cmax = jnp.max(cl, axis=1, keepdims=True)
        csum = jnp.sum(jnp.exp(cl - cmax), axis=1, keepdims=True)
        c_idx = _first_argmax(cl, cmax, N_CLUSTERS)
        tl = lax.dot_general(xb, p_ref[...], (((1,), (1,)), ((), ())),
                             preferred_element_type=_F32)
        tcol = (lax.broadcasted_iota(jnp.int32, tl.shape, 1)
                // TILES_PER_CLUSTER)
        tlm = jnp.where(tcol == c_idx, tl, _NEG)
        tmax = jnp.max(tlm, axis=1, keepdims=True)
        tsum = jnp.sum(jnp.exp(tlm - tmax), axis=1, keepdims=True)
        tidx_ref[...] = _first_argmax(tlm, tmax, NUM_TILES)
        gate_ref[...] = 1.0 / (csum * tsum)
        xh_ref[...] = (xb * (1.0 / (D_MODEL ** 0.5))).astype(jnp.bfloat16)

    xh = xh_ref[...]
    kc = k_ref[...].astype(jnp.bfloat16)
    gl = lax.dot_general(xh, kc, (((1,), (1,)), ((), ())),
                         preferred_element_type=_F32)
    tcol = (lax.broadcasted_iota(jnp.int32, gl.shape, 1) // GRID_SIZE
            + c * TILES_PER_CLUSTER)
    pr = jnp.where(tcol == tidx_ref[...],
                   jnp.exp(jnp.minimum(gl, 60.0)), 0.0)
    prh = pr.astype(jnp.bfloat16)
    ones = jnp.ones((CHUNK, 128), jnp.bfloat16)
    s = lax.dot_general(prh, ones, (((1,), (0,)), ((), ())),
                        preferred_element_type=_F32)[:, :1]
    y = lax.dot_general(prh, v_ref[...].astype(jnp.bfloat16),
                        (((1,), (0,)), ((), ())),
                        preferred_element_type=_F32)

    @pl.when(c == 0)
    def _init_acc():
        yacc_ref[...] = y
        sacc_ref[...] = s

    @pl.when(c > 0)
    def _accum():
        yacc_ref[...] += y
        sacc_ref[...] += s

    @pl.when(c == N_CLUSTERS - 1)
    def _finalize():
        o_ref[...] = (x_ref[...]
                      + (gate_ref[...] / jnp.maximum(sacc_ref[...], 1e-30))
                      * yacc_ref[...])


@jax.jit
def kernel(x, Wc, P, Kt, Vt):
    n, d = x.shape
    k2 = Kt.reshape(TG, d)
    v2 = Vt.reshape(TG, d)
    return pl.pallas_call(
        _body,
        grid=(N_CLUSTERS,),
        in_specs=[
            pl.BlockSpec((n, d), lambda c: (0, 0)),
            pl.BlockSpec((N_CLUSTERS, d), lambda c: (0, 0)),
            pl.BlockSpec((NUM_TILES, d), lambda c: (0, 0)),
            pl.BlockSpec((CHUNK, d), lambda c: (c, 0)),
            pl.BlockSpec((CHUNK, d), lambda c: (c, 0)),
        ],
        out_specs=pl.BlockSpec((n, d), lambda c: (0, 0)),
        out_shape=jax.ShapeDtypeStruct((n, d), jnp.float32),
        scratch_shapes=[
            pltpu.VMEM((n, d), jnp.bfloat16),
            pltpu.VMEM((n, 1), jnp.int32),
            pltpu.VMEM((n, 1), jnp.float32),
            pltpu.VMEM((n, d), jnp.float32),
            pltpu.VMEM((n, 1), jnp.float32),
        ],
        compiler_params=pltpu.CompilerParams(
            dimension_semantics=("arbitrary",),
        ),
    )(x, Wc, P, k2, v2)


# speedup vs baseline: 1.2744x; 1.0312x over previous
import jax
import jax.numpy as jnp
from jax import lax
from jax.experimental import pallas as pl
from jax.experimental.pallas import tpu as pltpu

D_MODEL = 1024
NUM_TILES = 64
TILES_PER_CLUSTER = 8
GRID_SIZE = 64
N_CLUSTERS = NUM_TILES // TILES_PER_CLUSTER
TG = NUM_TILES * GRID_SIZE
CHUNK = TG // N_CLUSTERS

_NEG = -1e30
_F32 = jnp.float32


def _first_argmax(vals, maxv, width):
    col = lax.broadcasted_iota(jnp.int32, vals.shape, 1)
    cand = jnp.where(vals >= maxv, col, jnp.int32(width))
    return jnp.min(cand, axis=1, keepdims=True)


def _body(x_ref, wc_ref, p_ref, k_ref, v_ref, o_ref,
          xh_ref, tidx_ref, gate_ref, yacc_ref, sacc_ref):
    c = pl.program_id(0)

    @pl.when(c == 0)
    def _routing():
        xb = x_ref[...]
        cl = lax.dot_general(xb, wc_ref[...], (((1,), (1,)), ((), ())),
                             preferred_element_type=_F32)
        cmax = jnp.max(cl, axis=1, keepdims=True)
        csum = jnp.sum(jnp.exp(cl - cmax), axis=1, keepdims=True)
        c_idx = _first_argmax(cl, cmax, N_CLUSTERS)
        tl = lax.dot_general(xb, p_ref[...], (((1,), (1,)), ((), ())),
                             preferred_element_type=_F32)
        tcol = (lax.broadcasted_iota(jnp.int32, tl.shape, 1)
                // TILES_PER_CLUSTER)
        tlm = jnp.where(tcol == c_idx, tl, _NEG)
        tmax = jnp.max(tlm, axis=1, keepdims=True)
        tsum = jnp.sum(jnp.exp(tlm - tmax), axis=1, keepdims=True)
        tidx_ref[...] = _first_argmax(tlm, tmax, NUM_TILES)
        gate_ref[...] = 1.0 / (csum * tsum)
        xh_ref[...] = (xb * (1.0 / (D_MODEL ** 0.5))).astype(jnp.bfloat16)

    xh = xh_ref[...]
    kc = k_ref[...].astype(jnp.bfloat16)
    gl = lax.dot_general(xh, kc, (((1,), (1,)), ((), ())),
                         preferred_element_type=_F32)
    tcol = (lax.broadcasted_iota(jnp.int32, gl.shape, 1) // GRID_SIZE
            + c * TILES_PER_CLUSTER)
    pr = jnp.where(tcol == tidx_ref[...],
                   jnp.exp(jnp.minimum(gl, 60.0)), 0.0)
    s = jnp.sum(pr, axis=1, keepdims=True)
    y = lax.dot_general(pr.astype(jnp.bfloat16), v_ref[...].astype(jnp.bfloat16),
                        (((1,), (0,)), ((), ())),
                        preferred_element_type=_F32)

    @pl.when(c == 0)
    def _init_acc():
        yacc_ref[...] = y
        sacc_ref[...] = s

    @pl.when(c > 0)
    def _accum():
        yacc_ref[...] += y
        sacc_ref[...] += s

    @pl.when(c == N_CLUSTERS - 1)
    def _finalize():
        o_ref[...] = (x_ref[...]
                      + (gate_ref[...] / jnp.maximum(sacc_ref[...], 1e-30))
                      * yacc_ref[...])


@jax.jit
def kernel(x, Wc, P, Kt, Vt):
    n, d = x.shape
    k2 = Kt.reshape(TG, d)
    v2 = Vt.reshape(TG, d)
    return pl.pallas_call(
        _body,
        grid=(N_CLUSTERS,),
        in_specs=[
            pl.BlockSpec((n, d), lambda c: (0, 0)),
            pl.BlockSpec((N_CLUSTERS, d), lambda c: (0, 0)),
            pl.BlockSpec((NUM_TILES, d), lambda c: (0, 0)),
            pl.BlockSpec((CHUNK, d), lambda c: (c, 0)),
            pl.BlockSpec((CHUNK, d), lambda c: (c, 0)),
        ],
        out_specs=pl.BlockSpec((n, d), lambda c: (0, 0)),
        out_shape=jax.ShapeDtypeStruct((n, d), jnp.float32),
        scratch_shapes=[
            pltpu.VMEM((n, d), jnp.bfloat16),
            pltpu.VMEM((n, 1), jnp.int32),
            pltpu.VMEM((n, 1), jnp.float32),
            pltpu.VMEM((n, d), jnp.float32),
            pltpu.VMEM((n, 1), jnp.float32),
        ],
        compiler_params=pltpu.CompilerParams(
            dimension_semantics=("arbitrary",),
        ),
    )(x, Wc, P, k2, v2)
